# baseline (speedup 1.0000x reference)
import functools

import jax
import jax.numpy as jnp
from jax import lax
from jax.experimental import pallas as pl
from jax.experimental.pallas import tpu as pltpu
from jax.experimental.pallas import tpu_sc as plsc

NC = 2
NS = 16
NW = NC * NS
L = 16

ROWS = 128
COLS = 32768
ROWS_PER_W = ROWS // NW
CHUNK = 8192
CHUNKS_PER_ROW = COLS // CHUNK
UNROLL = 8
ITERS = CHUNK // (L * UNROLL)

_mesh = plsc.VectorSubcoreMesh(core_axis_name="c", subcore_axis_name="s")

NEG_INF = float("-inf")
BIG_I32 = 2**31 - 1


@functools.partial(
    pl.kernel,
    out_type=jax.ShapeDtypeStruct((NW, L), jnp.int32),
    mesh=_mesh,
    scratch_types=[
        pltpu.VMEM((2, CHUNK), jnp.float32),
        pltpu.VMEM((L,), jnp.int32),
        pltpu.SemaphoreType.DMA,
        pltpu.SemaphoreType.DMA,
    ],
)
def _argmax_sc(logits_hbm, out_hbm, buf, res, sem0, sem1):
    wid = lax.axis_index("s") * NC + lax.axis_index("c")
    sems = (sem0, sem1)
    lane = lax.iota(jnp.int32, L)

    def start_dma(i, slot):
        r, c = divmod(i, CHUNKS_PER_ROW)
        row = wid * ROWS_PER_W + r
        return pltpu.async_copy(
            logits_hbm.at[row, pl.ds(c * CHUNK, CHUNK)],
            buf.at[slot],
            sems[slot],
        )

    n_chunks = ROWS_PER_W * CHUNKS_PER_ROW
    cps = [None, None]
    cps[0] = start_dma(0, 0)

    res_vec = jnp.zeros((L,), jnp.int32)
    m = jnp.full((L,), NEG_INF, jnp.float32)
    mi = jnp.zeros((L,), jnp.int32)
    iv = lane

    for i in range(n_chunks):
        slot = i & 1
        if i + 1 < n_chunks:
            cps[1 - slot] = start_dma(i + 1, 1 - slot)
        cps[slot].wait()

        def body(k, carry):
            m, mi, iv = carry
            base = k * (L * UNROLL)
            for j in range(UNROLL):
                v = buf[slot, pl.ds(base + j * L, L)]
                pred = v > m
                m = jnp.where(pred, v, m)
                mi = jnp.where(pred, iv, mi)
                iv = iv + L
            return m, mi, iv

        m, mi, iv = lax.fori_loop(0, ITERS, body, (m, mi, iv))

        if i % CHUNKS_PER_ROW == CHUNKS_PER_ROW - 1:
            r = i // CHUNKS_PER_ROW
            ks = [m[j] for j in range(L)]
            is_ = [mi[j] for j in range(L)]
            while len(ks) > 1:
                nk, ni = [], []
                for a in range(0, len(ks), 2):
                    k0, i0 = ks[a], is_[a]
                    k1, i1 = ks[a + 1], is_[a + 1]
                    better = (k1 > k0) | ((k1 == k0) & (i1 < i0))
                    nk.append(jnp.where(better, k1, k0))
                    ni.append(jnp.where(better, i1, i0))
                ks, is_ = nk, ni
            res_vec = jnp.where(lane == r, is_[0], res_vec)
            m = jnp.full((L,), NEG_INF, jnp.float32)
            mi = jnp.zeros((L,), jnp.int32)
            iv = lane

    res[...] = res_vec
    pltpu.sync_copy(res, out_hbm.at[wid])


def kernel(logits):
    staged = _argmax_sc(logits)
    return staged[:, :ROWS_PER_W].reshape(ROWS, 1)


# speedup vs baseline: 1.1732x; 1.1732x over previous
import functools

import jax
import jax.numpy as jnp
from jax import lax
from jax.experimental import pallas as pl
from jax.experimental.pallas import tpu as pltpu
from jax.experimental.pallas import tpu_sc as plsc

NC = 2
NS = 16
NW = NC * NS
L = 16

ROWS = 128
COLS = 32768
ROWS_PER_W = ROWS // NW
GV = 8
GELEMS = GV * L
GROUPS = COLS // GELEMS
UNROLL_G = 4
ITERS = GROUPS // UNROLL_G

NEG_INF = float("-inf")


def _tournament(ks, is_):
    while len(ks) > 1:
        nk, ni = [], []
        for a in range(0, len(ks), 2):
            k0, i0 = ks[a], is_[a]
            k1, i1 = ks[a + 1], is_[a + 1]
            better = (k1 > k0) | ((k1 == k0) & (i1 < i0))
            nk.append(jnp.where(better, k1, k0))
            ni.append(jnp.where(better, i1, i0))
        ks, is_ = nk, ni
    return ks[0], is_[0]


@functools.partial(
    pl.kernel,
    out_type=jax.ShapeDtypeStruct((NW, L), jnp.int32),
    mesh=plsc.VectorSubcoreMesh(core_axis_name="c", subcore_axis_name="s"),
    scratch_types=[
        pltpu.VMEM((2, COLS), jnp.float32),
        pltpu.VMEM((L,), jnp.int32),
        pltpu.SemaphoreType.DMA,
        pltpu.SemaphoreType.DMA,
    ],
)
def _argmax_sc(logits_hbm, out_hbm, buf, res, sem0, sem1):
    wid = lax.axis_index("s") * NC + lax.axis_index("c")
    sems = (sem0, sem1)
    lane = lax.iota(jnp.int32, L)

    def start_dma(r, slot):
        return pltpu.async_copy(
            logits_hbm.at[wid * ROWS_PER_W + r], buf.at[slot], sems[slot]
        )

    cps = [None, None]
    cps[0] = start_dma(0, 0)
    res_vec = jnp.zeros((L,), jnp.int32)

    for r in range(ROWS_PER_W):
        slot = r & 1
        if r + 1 < ROWS_PER_W:
            cps[1 - slot] = start_dma(r + 1, 1 - slot)
        cps[slot].wait()

        def body(k, carry):
            m, bb = carry
            g0 = k * UNROLL_G
            for u in range(UNROLL_G):
                base = (g0 + u) * GELEMS
                v = [buf[slot, pl.ds(base + j * L, L)] for j in range(GV)]
                while len(v) > 1:
                    v = [jnp.maximum(v[a], v[a + 1]) for a in range(0, len(v), 2)]
                pred = v[0] > m
                m = jnp.where(pred, v[0], m)
                bb = jnp.where(pred, g0 + u, bb)
            return m, bb

        m0 = jnp.full((L,), NEG_INF, jnp.float32)
        b0 = jnp.zeros((L,), jnp.int32)
        m, bb = lax.fori_loop(0, ITERS, body, (m0, b0))

        _, gstar = _tournament(
            [m[j] for j in range(L)], [bb[j] for j in range(L)]
        )

        gbase = gstar * GELEMS
        ivbase = gbase + lane
        fm = jnp.full((L,), NEG_INF, jnp.float32)
        fi = jnp.zeros((L,), jnp.int32)
        for j in range(GV):
            v = buf[slot, pl.ds(gbase + j * L, L)]
            pred = v > fm
            fm = jnp.where(pred, v, fm)
            fi = jnp.where(pred, ivbase + j * L, fi)
        _, ridx = _tournament([fm[j] for j in range(L)], [fi[j] for j in range(L)])
        res_vec = jnp.where(lane == r, ridx, res_vec)

    res[...] = res_vec
    pltpu.sync_copy(res, out_hbm.at[wid])


def kernel(logits):
    staged = _argmax_sc(logits)
    return staged[:, :ROWS_PER_W].reshape(ROWS, 1)
